# baseline (speedup 1.0000x reference)
import jax
import jax.numpy as jnp
from jax.experimental import pallas as pl

N, D, HID, OUT, _BN = 10000, 128, 256, 64, 1000


def _mlp_body(a_ref, W1_ref, b1_ref, W2_ref, b2_ref, out_ref):
    z = jnp.dot(a_ref[...], W1_ref[...], preferred_element_type=jnp.float32)
    z = jnp.maximum(z + b1_ref[...], 0.0)
    o = jnp.dot(z, W2_ref[...], preferred_element_type=jnp.float32)
    o = o + b2_ref[...]
    nrm = jnp.sqrt(jnp.sum(o * o, axis=1, keepdims=True))
    out_ref[...] = o / jnp.maximum(nrm, 1e-12)


def _conv(h, ei):
    msgs = jnp.take(h, ei[0], axis=0)
    agg = jax.ops.segment_sum(msgs, ei[1], num_segments=N)
    deg = jax.ops.segment_sum(jnp.ones((ei.shape[1],), h.dtype), ei[1],
                              num_segments=N)
    return agg / jnp.maximum(deg, 1.0)[:, None]


def kernel(x, edge_index1, edge_index2, W1, b1, W2, b2):
    h1 = _conv(x, edge_index1)
    h2 = _conv(h1, edge_index2)
    a = jnp.concatenate([h1, h2], axis=1)
    mlp = pl.pallas_call(
        _mlp_body,
        grid=(N // _BN,),
        in_specs=[
            pl.BlockSpec((_BN, 2 * D), lambda i: (i, 0)),
            pl.BlockSpec((2 * D, HID), lambda i: (0, 0)),
            pl.BlockSpec((1, HID), lambda i: (0, 0)),
            pl.BlockSpec((HID, OUT), lambda i: (0, 0)),
            pl.BlockSpec((1, OUT), lambda i: (0, 0)),
        ],
        out_specs=pl.BlockSpec((_BN, OUT), lambda i: (i, 0)),
        out_shape=jax.ShapeDtypeStruct((N, OUT), jnp.float32),
    )
    return mlp(a, W1, b1.reshape(1, HID), W2, b2.reshape(1, OUT))


# speedup vs baseline: 1.3185x; 1.3185x over previous
import functools

import jax
import jax.numpy as jnp
from jax import lax
from jax.experimental import pallas as pl
from jax.experimental.pallas import tpu as pltpu
from jax.experimental.pallas import tpu_sc as plsc

N = 10000
E = 320000
D = 128
HID = 256
OUT = 64

NC = 2
NS = 16
NW = NC * NS
EPW = E // NW
CH = 80
NCH = EPW // CH
DR = 80
LANES = 16

RLO = (0, 2560, 5120, 7680)
RHI = (2560, 5120, 7680, 10000)
RMAX = 2560
ACCR = RMAX + 8
DGR = 1280


def _hop_body(table, src, dst, i8, acc_out, deg_out,
              src_c, dst_c, sel_c, gsel_c, ssel_c, rows_v,
              acc_sp, deg_sp, i8_sp, sem):
    cid = lax.axis_index("c")
    sid = lax.axis_index("s")
    wid = sid * NC + cid

    zeros16 = jnp.zeros((LANES,), jnp.float32)

    def _fill(r, carry):
        for c in range(D // LANES):
            rows_v[r, pl.ds(c * LANES, LANES)] = zeros16
        return carry
    lax.fori_loop(0, CH, _fill, 0)

    @pl.when(sid == 0)
    def _stage_i8():
        pltpu.sync_copy(i8, i8_sp)

    base = wid * EPW

    for p in range(len(RLO)):
        lo, hi = RLO[p], RHI[p]
        rsize = hi - lo
        ndr = rsize // DR
        cpt = (ndr + NS - 1) // NS

        for k in range(cpt):
            c = sid + k * NS

            @pl.when(c < ndr)
            def _zero_chunk():
                r0 = pl.multiple_of(c * DR, 8)
                pltpu.sync_copy(rows_v, acc_sp.at[pl.ds(r0, DR)])
        if p == 0:
            d0 = pl.multiple_of(sid * DR, 8)
            pltpu.sync_copy(rows_v, deg_sp.at[pl.ds(d0, DR)])
        plsc.subcore_barrier()

        def _chunk(c, carry):
            off = pl.multiple_of(base + c * CH, 8)
            pltpu.sync_copy(src.at[pl.ds(off, CH)], src_c)
            pltpu.sync_copy(dst.at[pl.ds(off, CH)], dst_c)
            ca = pltpu.async_copy(table.at[src_c], rows_v, sem)
            for j in range(CH // LANES):
                dv = dst_c[pl.ds(j * LANES, LANES)]
                in_r = jnp.logical_and(dv >= lo, dv < hi)
                sel_c[pl.ds(j * LANES, LANES)] = jnp.where(in_r, dv - lo, RMAX)
                if p == 0:
                    gsel_c[pl.ds(j * LANES, LANES)] = dv & 7
                    ssel_c[pl.ds(j * LANES, LANES)] = dv >> 3
            ca.wait()
            pltpu.sync_copy(rows_v, acc_sp.at[sel_c], add=True)
            if p == 0:
                pltpu.async_copy(i8_sp.at[gsel_c], rows_v, sem).wait()
                pltpu.sync_copy(rows_v, deg_sp.at[ssel_c], add=True)
            return carry
        lax.fori_loop(0, NCH, _chunk, 0)

        plsc.subcore_barrier()
        for k in range(cpt):
            c = sid + k * NS

            @pl.when(c < ndr)
            def _drain_chunk():
                r0 = pl.multiple_of(c * DR, 8)
                g0 = pl.multiple_of(lo + c * DR, 8)
                pltpu.sync_copy(acc_sp.at[pl.ds(r0, DR)], rows_v)
                pltpu.sync_copy(rows_v, acc_out.at[cid, pl.ds(g0, DR)])
        if p == 0:
            d0 = pl.multiple_of(sid * DR, 8)
            pltpu.sync_copy(deg_sp.at[pl.ds(d0, DR)], rows_v)
            pltpu.sync_copy(rows_v, deg_out.at[cid, pl.ds(d0, DR)])
        plsc.subcore_barrier()

        def _rezero(r, carry):
            for c in range(D // LANES):
                rows_v[r, pl.ds(c * LANES, LANES)] = zeros16
            return carry
        if p + 1 < len(RLO):
            lax.fori_loop(0, CH, _rezero, 0)


def _make_hop():
    mesh = plsc.VectorSubcoreMesh(core_axis_name="c", subcore_axis_name="s")
    return functools.partial(
        pl.kernel,
        out_type=[
            jax.ShapeDtypeStruct((NC, N, D), jnp.float32),
            jax.ShapeDtypeStruct((NC, DGR, D), jnp.float32),
        ],
        mesh=mesh,
        scratch_types=[
            pltpu.VMEM((CH,), jnp.int32),
            pltpu.VMEM((CH,), jnp.int32),
            pltpu.VMEM((CH,), jnp.int32),
            pltpu.VMEM((CH,), jnp.int32),
            pltpu.VMEM((CH,), jnp.int32),
            pltpu.VMEM((CH, D), jnp.float32),
            pltpu.VMEM_SHARED((ACCR, D), jnp.float32),
            pltpu.VMEM_SHARED((DGR, D), jnp.float32),
            pltpu.VMEM_SHARED((8, D), jnp.float32),
            pltpu.SemaphoreType.DMA,
        ],
    )(_hop_body)


_BN = 1000


def _combine_body(acc_ref, deg_ref, out_ref):
    s = acc_ref[0] + acc_ref[1]
    dsum = deg_ref[0, :, 0] + deg_ref[1, :, 0]
    out_ref[...] = s / jnp.maximum(dsum, 1.0)[:, None]


def _mlp_body(h1_ref, acc_ref, deg_ref, W1_ref, b1_ref, W2_ref, b2_ref,
              out_ref):
    s = acc_ref[0] + acc_ref[1]
    dsum = deg_ref[0, :, 0] + deg_ref[1, :, 0]
    h2 = s / jnp.maximum(dsum, 1.0)[:, None]
    a = jnp.concatenate([h1_ref[...], h2], axis=1)
    z = jnp.dot(a, W1_ref[...], preferred_element_type=jnp.float32)
    z = jnp.maximum(z + b1_ref[...], 0.0)
    o = jnp.dot(z, W2_ref[...], preferred_element_type=jnp.float32)
    o = o + b2_ref[...]
    nrm = jnp.sqrt(jnp.sum(o * o, axis=1, keepdims=True))
    out_ref[...] = o / jnp.maximum(nrm, 1e-12)


def _unpack_deg(deg_packed):
    d = deg_packed.reshape(NC, DGR, 8, LANES)[:, :, :, 0]
    return d.reshape(NC, DGR * 8)[:, :N].reshape(NC, N, 1)


def kernel(x, edge_index1, edge_index2, W1, b1, W2, b2):
    hop = _make_hop()
    i8 = jnp.repeat(jnp.eye(8, dtype=jnp.float32), LANES, axis=1)

    acc1, deg1p = hop(x, edge_index1[0], edge_index1[1], i8)
    deg1 = _unpack_deg(deg1p)

    combine = pl.pallas_call(
        _combine_body,
        grid=(N // _BN,),
        in_specs=[
            pl.BlockSpec((NC, _BN, D), lambda i: (0, i, 0)),
            pl.BlockSpec((NC, _BN, 1), lambda i: (0, i, 0)),
        ],
        out_specs=pl.BlockSpec((_BN, D), lambda i: (i, 0)),
        out_shape=jax.ShapeDtypeStruct((N, D), jnp.float32),
    )
    h1 = combine(acc1, deg1)

    acc2, deg2p = hop(h1, edge_index2[0], edge_index2[1], i8)
    deg2 = _unpack_deg(deg2p)

    mlp = pl.pallas_call(
        _mlp_body,
        grid=(N // _BN,),
        in_specs=[
            pl.BlockSpec((_BN, D), lambda i: (i, 0)),
            pl.BlockSpec((NC, _BN, D), lambda i: (0, i, 0)),
            pl.BlockSpec((NC, _BN, 1), lambda i: (0, i, 0)),
            pl.BlockSpec((2 * D, HID), lambda i: (0, 0)),
            pl.BlockSpec((1, HID), lambda i: (0, 0)),
            pl.BlockSpec((HID, OUT), lambda i: (0, 0)),
            pl.BlockSpec((1, OUT), lambda i: (0, 0)),
        ],
        out_specs=pl.BlockSpec((_BN, OUT), lambda i: (i, 0)),
        out_shape=jax.ShapeDtypeStruct((N, OUT), jnp.float32),
    )
    return mlp(h1, acc2, deg2, W1, b1.reshape(1, HID), W2, b2.reshape(1, OUT))


# speedup vs baseline: 1.6725x; 1.2685x over previous
import functools

import jax
import jax.numpy as jnp
from jax import lax
from jax.experimental import pallas as pl
from jax.experimental.pallas import tpu as pltpu
from jax.experimental.pallas import tpu_sc as plsc

N = 10000
E = 320000
D = 128
HID = 256
OUT = 64

NC = 2
NS = 16
NW = NC * NS
EPW = E // NW
CH = 80
NCH = EPW // CH
DR = 80
LANES = 16

RLO = (0, 3360, 6720)
RHI = (3360, 6720, 10000)
RMAX = 3360
ACCR = RMAX + 8
DGR = 640


def _hop_body(table, src, dst, i8, acc_out, deg_out,
              src_c, dst_c, sel_c, gsel_c, ssel_c, rows_v,
              acc_sp, deg_sp, i8_sp, sem):
    cid = lax.axis_index("c")
    sid = lax.axis_index("s")
    wid = sid * NC + cid

    zeros16 = jnp.zeros((LANES,), jnp.float32)

    def _fill(r, carry):
        for c in range(D // LANES):
            rows_v[r, pl.ds(c * LANES, LANES)] = zeros16
        return carry
    lax.fori_loop(0, CH, _fill, 0)

    @pl.when(sid == 0)
    def _stage_i8():
        pltpu.sync_copy(i8, i8_sp)

    base = wid * EPW

    for p in range(len(RLO)):
        lo, hi = RLO[p], RHI[p]
        rsize = hi - lo
        ndr = rsize // DR
        cpt = (ndr + NS - 1) // NS

        for k in range(cpt):
            c = sid + k * NS

            @pl.when(c < ndr)
            def _zero_chunk():
                r0 = pl.multiple_of(c * DR, 8)
                pltpu.sync_copy(rows_v, acc_sp.at[pl.ds(r0, DR)])
        if p == 0:
            @pl.when(sid < DGR // DR)
            def _zero_deg():
                d0 = pl.multiple_of(sid * DR, 8)
                pltpu.sync_copy(rows_v, deg_sp.at[pl.ds(d0, DR)])
        plsc.subcore_barrier()

        def _chunk(c, carry):
            off = pl.multiple_of(base + c * CH, 8)
            pltpu.sync_copy(src.at[pl.ds(off, CH)], src_c)
            pltpu.sync_copy(dst.at[pl.ds(off, CH)], dst_c)
            ca = pltpu.async_copy(table.at[src_c], rows_v, sem)
            for j in range(CH // LANES):
                dv = dst_c[pl.ds(j * LANES, LANES)]
                in_r = jnp.logical_and(dv >= lo, dv < hi)
                sel_c[pl.ds(j * LANES, LANES)] = jnp.where(in_r, dv - lo, RMAX)
                if p == 0:
                    gsel_c[pl.ds(j * LANES, LANES)] = dv & 15
                    ssel_c[pl.ds(j * LANES, LANES)] = dv >> 4
            ca.wait()
            pltpu.sync_copy(rows_v, acc_sp.at[sel_c], add=True)
            if p == 0:
                pltpu.async_copy(i8_sp.at[gsel_c], rows_v, sem).wait()
                pltpu.sync_copy(rows_v, deg_sp.at[ssel_c], add=True)
            return carry
        lax.fori_loop(0, NCH, _chunk, 0)

        plsc.subcore_barrier()
        for k in range(cpt):
            c = sid + k * NS

            @pl.when(c < ndr)
            def _drain_chunk():
                r0 = pl.multiple_of(c * DR, 8)
                g0 = pl.multiple_of(lo + c * DR, 8)
                pltpu.sync_copy(acc_sp.at[pl.ds(r0, DR)], rows_v)
                pltpu.sync_copy(rows_v, acc_out.at[cid, pl.ds(g0, DR)])
        if p == 0:
            @pl.when(sid < DGR // DR)
            def _drain_deg():
                d0 = pl.multiple_of(sid * DR, 8)
                pltpu.sync_copy(deg_sp.at[pl.ds(d0, DR)], rows_v)
                pltpu.sync_copy(rows_v, deg_out.at[cid, pl.ds(d0, DR)])
        plsc.subcore_barrier()

        def _rezero(r, carry):
            for c in range(D // LANES):
                rows_v[r, pl.ds(c * LANES, LANES)] = zeros16
            return carry
        if p + 1 < len(RLO):
            lax.fori_loop(0, CH, _rezero, 0)


def _make_hop():
    mesh = plsc.VectorSubcoreMesh(core_axis_name="c", subcore_axis_name="s")
    return functools.partial(
        pl.kernel,
        out_type=[
            jax.ShapeDtypeStruct((NC, N, D), jnp.float32),
            jax.ShapeDtypeStruct((NC, DGR, D), jnp.float32),
        ],
        mesh=mesh,
        scratch_types=[
            pltpu.VMEM((CH,), jnp.int32),
            pltpu.VMEM((CH,), jnp.int32),
            pltpu.VMEM((CH,), jnp.int32),
            pltpu.VMEM((CH,), jnp.int32),
            pltpu.VMEM((CH,), jnp.int32),
            pltpu.VMEM((CH, D), jnp.float32),
            pltpu.VMEM_SHARED((ACCR, D), jnp.float32),
            pltpu.VMEM_SHARED((DGR, D), jnp.float32),
            pltpu.VMEM_SHARED((16, D), jnp.float32),
            pltpu.SemaphoreType.DMA,
        ],
    )(_hop_body)


_BN = 1000


def _combine_body(acc_ref, deg_ref, out_ref):
    s = acc_ref[0] + acc_ref[1]
    dsum = deg_ref[0, :, 0] + deg_ref[1, :, 0]
    out_ref[...] = s / jnp.maximum(dsum, 1.0)[:, None]


def _mlp_body(h1_ref, acc_ref, deg_ref, W1_ref, b1_ref, W2_ref, b2_ref,
              out_ref):
    s = acc_ref[0] + acc_ref[1]
    dsum = deg_ref[0, :, 0] + deg_ref[1, :, 0]
    h2 = s / jnp.maximum(dsum, 1.0)[:, None]
    a = jnp.concatenate([h1_ref[...], h2], axis=1)
    z = jnp.dot(a, W1_ref[...], preferred_element_type=jnp.float32)
    z = jnp.maximum(z + b1_ref[...], 0.0)
    o = jnp.dot(z, W2_ref[...], preferred_element_type=jnp.float32)
    o = o + b2_ref[...]
    nrm = jnp.sqrt(jnp.sum(o * o, axis=1, keepdims=True))
    out_ref[...] = o / jnp.maximum(nrm, 1e-12)


def _unpack_deg(deg_packed):
    d = deg_packed.reshape(NC, DGR, 16, 8)[:, :, :, 0]
    return d.reshape(NC, DGR * 16)[:, :N].reshape(NC, N, 1)


def kernel(x, edge_index1, edge_index2, W1, b1, W2, b2):
    hop = _make_hop()
    i8 = jnp.repeat(jnp.eye(16, dtype=jnp.float32), 8, axis=1)

    acc1, deg1p = hop(x, edge_index1[0], edge_index1[1], i8)
    deg1 = _unpack_deg(deg1p)

    combine = pl.pallas_call(
        _combine_body,
        grid=(N // _BN,),
        in_specs=[
            pl.BlockSpec((NC, _BN, D), lambda i: (0, i, 0)),
            pl.BlockSpec((NC, _BN, 1), lambda i: (0, i, 0)),
        ],
        out_specs=pl.BlockSpec((_BN, D), lambda i: (i, 0)),
        out_shape=jax.ShapeDtypeStruct((N, D), jnp.float32),
    )
    h1 = combine(acc1, deg1)

    acc2, deg2p = hop(h1, edge_index2[0], edge_index2[1], i8)
    deg2 = _unpack_deg(deg2p)

    mlp = pl.pallas_call(
        _mlp_body,
        grid=(N // _BN,),
        in_specs=[
            pl.BlockSpec((_BN, D), lambda i: (i, 0)),
            pl.BlockSpec((NC, _BN, D), lambda i: (0, i, 0)),
            pl.BlockSpec((NC, _BN, 1), lambda i: (0, i, 0)),
            pl.BlockSpec((2 * D, HID), lambda i: (0, 0)),
            pl.BlockSpec((1, HID), lambda i: (0, 0)),
            pl.BlockSpec((HID, OUT), lambda i: (0, 0)),
            pl.BlockSpec((1, OUT), lambda i: (0, 0)),
        ],
        out_specs=pl.BlockSpec((_BN, OUT), lambda i: (i, 0)),
        out_shape=jax.ShapeDtypeStruct((N, OUT), jnp.float32),
    )
    return mlp(h1, acc2, deg2, W1, b1.reshape(1, HID), W2, b2.reshape(1, OUT))


# speedup vs baseline: 1.8765x; 1.1220x over previous
import functools

import jax
import jax.numpy as jnp
from jax import lax
from jax.experimental import pallas as pl
from jax.experimental.pallas import tpu as pltpu
from jax.experimental.pallas import tpu_sc as plsc

N = 10000
E = 320000
D = 128
HID = 256
OUT = 64

NC = 2
NS = 16
NW = NC * NS
EPW = E // NW
CH = 80
NCH = EPW // CH
DR = 80
LANES = 16

RLO = (0, 3360, 6720)
RHI = (3360, 6720, 10000)
RMAX = 3360
ACCR = RMAX + 8
DGR = 640


def _hop_body(table, eidx, i8, acc_out, deg_out,
              ei_c, sel_c, gsel_c, ssel_c, rows_v,
              acc_sp, deg_sp, i8_sp, sem):
    cid = lax.axis_index("c")
    sid = lax.axis_index("s")
    wid = sid * NC + cid

    zeros16 = jnp.zeros((LANES,), jnp.float32)

    def _fill(r, carry):
        for c in range(D // LANES):
            rows_v[r, pl.ds(c * LANES, LANES)] = zeros16
        return carry
    lax.fori_loop(0, CH, _fill, 0)

    @pl.when(sid == 0)
    def _stage_i8():
        pltpu.sync_copy(i8, i8_sp)

    base = wid * EPW * 2

    for p in range(len(RLO)):
        lo, hi = RLO[p], RHI[p]
        rsize = hi - lo
        ndr = rsize // DR
        cpt = (ndr + NS - 1) // NS

        for k in range(cpt):
            c = sid + k * NS

            @pl.when(c < ndr)
            def _zero_chunk():
                r0 = pl.multiple_of(c * DR, 8)
                pltpu.sync_copy(rows_v, acc_sp.at[pl.ds(r0, DR)])
        if p == 0:
            @pl.when(sid < DGR // DR)
            def _zero_deg():
                d0 = pl.multiple_of(sid * DR, 8)
                pltpu.sync_copy(rows_v, deg_sp.at[pl.ds(d0, DR)])
        plsc.subcore_barrier()

        def _chunk(c, carry):
            off = pl.multiple_of(base + c * (2 * CH), 8)
            pltpu.sync_copy(eidx.at[pl.ds(off, 2 * CH)], ei_c)
            for j in range(CH // LANES):
                dv = ei_c[pl.ds(CH + j * LANES, LANES)]
                in_r = jnp.logical_and(dv >= lo, dv < hi)
                sel_c[pl.ds(j * LANES, LANES)] = jnp.where(in_r, dv - lo, RMAX)
                if p == 0:
                    gsel_c[pl.ds(j * LANES, LANES)] = dv & 15
                    ssel_c[pl.ds(j * LANES, LANES)] = dv >> 4
            ca = pltpu.async_copy(table.at[ei_c.at[pl.ds(0, CH)]], rows_v, sem)
            ca.wait()
            pltpu.sync_copy(rows_v, acc_sp.at[sel_c], add=True)
            if p == 0:
                pltpu.async_copy(i8_sp.at[gsel_c], rows_v, sem).wait()
                pltpu.sync_copy(rows_v, deg_sp.at[ssel_c], add=True)
            return carry
        lax.fori_loop(0, NCH, _chunk, 0)

        plsc.subcore_barrier()
        for k in range(cpt):
            c = sid + k * NS

            @pl.when(c < ndr)
            def _drain_chunk():
                r0 = pl.multiple_of(c * DR, 8)
                g0 = pl.multiple_of(lo + c * DR, 8)
                pltpu.sync_copy(acc_sp.at[pl.ds(r0, DR)], rows_v)
                pltpu.sync_copy(rows_v, acc_out.at[cid, pl.ds(g0, DR)])
        if p == 0:
            @pl.when(sid < DGR // DR)
            def _drain_deg():
                d0 = pl.multiple_of(sid * DR, 8)
                pltpu.sync_copy(deg_sp.at[pl.ds(d0, DR)], rows_v)
                pltpu.sync_copy(rows_v, deg_out.at[cid, pl.ds(d0, DR)])
        plsc.subcore_barrier()

        def _rezero(r, carry):
            for c in range(D // LANES):
                rows_v[r, pl.ds(c * LANES, LANES)] = zeros16
            return carry
        if p + 1 < len(RLO):
            lax.fori_loop(0, CH, _rezero, 0)


def _make_hop():
    mesh = plsc.VectorSubcoreMesh(core_axis_name="c", subcore_axis_name="s")
    return functools.partial(
        pl.kernel,
        out_type=[
            jax.ShapeDtypeStruct((NC, N, D), jnp.float32),
            jax.ShapeDtypeStruct((NC, DGR, D), jnp.float32),
        ],
        mesh=mesh,
        scratch_types=[
            pltpu.VMEM((2 * CH,), jnp.int32),
            pltpu.VMEM((CH,), jnp.int32),
            pltpu.VMEM((CH,), jnp.int32),
            pltpu.VMEM((CH,), jnp.int32),
            pltpu.VMEM((CH, D), jnp.float32),
            pltpu.VMEM_SHARED((ACCR, D), jnp.float32),
            pltpu.VMEM_SHARED((DGR, D), jnp.float32),
            pltpu.VMEM_SHARED((16, D), jnp.float32),
            pltpu.SemaphoreType.DMA,
        ],
    )(_hop_body)


_BN = 1000


def _combine_body(acc_ref, deg_ref, out_ref):
    s = acc_ref[0] + acc_ref[1]
    dsum = deg_ref[0, :, 0] + deg_ref[1, :, 0]
    out_ref[...] = s / jnp.maximum(dsum, 1.0)[:, None]


def _mlp_body(h1_ref, acc_ref, deg_ref, W1_ref, b1_ref, W2_ref, b2_ref,
              out_ref):
    s = acc_ref[0] + acc_ref[1]
    dsum = deg_ref[0, :, 0] + deg_ref[1, :, 0]
    h2 = s / jnp.maximum(dsum, 1.0)[:, None]
    a = jnp.concatenate([h1_ref[...], h2], axis=1)
    z = jnp.dot(a, W1_ref[...], preferred_element_type=jnp.float32)
    z = jnp.maximum(z + b1_ref[...], 0.0)
    o = jnp.dot(z, W2_ref[...], preferred_element_type=jnp.float32)
    o = o + b2_ref[...]
    nrm = jnp.sqrt(jnp.sum(o * o, axis=1, keepdims=True))
    out_ref[...] = o / jnp.maximum(nrm, 1e-12)


def _unpack_deg(deg_packed):
    d = deg_packed.reshape(NC, DGR, 16, 8)[:, :, :, 0]
    return d.reshape(NC, DGR * 16)[:, :N].reshape(NC, N, 1)


def _interleave(ei):
    sidx = ei[0].reshape(NW, NCH, 1, CH)
    didx = ei[1].reshape(NW, NCH, 1, CH)
    return jnp.concatenate([sidx, didx], axis=2).reshape(-1)


def kernel(x, edge_index1, edge_index2, W1, b1, W2, b2):
    hop = _make_hop()
    i8 = jnp.repeat(jnp.eye(16, dtype=jnp.float32), 8, axis=1)

    acc1, deg1p = hop(x, _interleave(edge_index1), i8)
    deg1 = _unpack_deg(deg1p)

    combine = pl.pallas_call(
        _combine_body,
        grid=(N // _BN,),
        in_specs=[
            pl.BlockSpec((NC, _BN, D), lambda i: (0, i, 0)),
            pl.BlockSpec((NC, _BN, 1), lambda i: (0, i, 0)),
        ],
        out_specs=pl.BlockSpec((_BN, D), lambda i: (i, 0)),
        out_shape=jax.ShapeDtypeStruct((N, D), jnp.float32),
    )
    h1 = combine(acc1, deg1)

    acc2, deg2p = hop(h1, _interleave(edge_index2), i8)
    deg2 = _unpack_deg(deg2p)

    mlp = pl.pallas_call(
        _mlp_body,
        grid=(N // _BN,),
        in_specs=[
            pl.BlockSpec((_BN, D), lambda i: (i, 0)),
            pl.BlockSpec((NC, _BN, D), lambda i: (0, i, 0)),
            pl.BlockSpec((NC, _BN, 1), lambda i: (0, i, 0)),
            pl.BlockSpec((2 * D, HID), lambda i: (0, 0)),
            pl.BlockSpec((1, HID), lambda i: (0, 0)),
            pl.BlockSpec((HID, OUT), lambda i: (0, 0)),
            pl.BlockSpec((1, OUT), lambda i: (0, 0)),
        ],
        out_specs=pl.BlockSpec((_BN, OUT), lambda i: (i, 0)),
        out_shape=jax.ShapeDtypeStruct((N, OUT), jnp.float32),
    )
    return mlp(h1, acc2, deg2, W1, b1.reshape(1, HID), W2, b2.reshape(1, OUT))


# speedup vs baseline: 2.1397x; 1.1402x over previous
import functools

import jax
import jax.numpy as jnp
from jax import lax
from jax.experimental import pallas as pl
from jax.experimental.pallas import tpu as pltpu
from jax.experimental.pallas import tpu_sc as plsc

N = 10000
E = 320000
D = 128
HID = 256
OUT = 64

NC = 2
NS = 16
NW = NC * NS
EPW = E // NW
CH = 80
NCH = EPW // CH
SB = 5
DR = 80
LANES = 16

RLO = (0, 3360, 6720)
RHI = (3360, 6720, 10000)
RMAX = 3360
ACCR = RMAX + 8
DGR = 640


def _hop_body(table, eidx, i8, acc_out, deg_out,
              ei_c, sel_c, gsel_c, ssel_c, rows_v,
              acc_sp, deg_sp, i8_sp, sem):
    cid = lax.axis_index("c")
    sid = lax.axis_index("s")
    wid = sid * NC + cid

    zeros16 = jnp.zeros((LANES,), jnp.float32)

    def _fill(r, carry):
        for c in range(D // LANES):
            rows_v[r, pl.ds(c * LANES, LANES)] = zeros16
        return carry
    lax.fori_loop(0, CH, _fill, 0)

    @pl.when(sid == 0)
    def _stage_i8():
        pltpu.sync_copy(i8, i8_sp)

    base = wid * EPW * 2

    for p in range(len(RLO)):
        lo, hi = RLO[p], RHI[p]
        rsize = hi - lo
        ndr = rsize // DR
        cpt = (ndr + NS - 1) // NS

        for k in range(cpt):
            c = sid + k * NS

            @pl.when(c < ndr)
            def _zero_chunk():
                r0 = pl.multiple_of(c * DR, 8)
                pltpu.sync_copy(rows_v, acc_sp.at[pl.ds(r0, DR)])
        if p == 0:
            @pl.when(sid < DGR // DR)
            def _zero_deg():
                d0 = pl.multiple_of(sid * DR, 8)
                pltpu.sync_copy(rows_v, deg_sp.at[pl.ds(d0, DR)])
        plsc.subcore_barrier()

        def _super(u, carry):
            off = pl.multiple_of(base + u * (SB * 2 * CH), 8)
            pltpu.sync_copy(eidx.at[pl.ds(off, SB * 2 * CH)], ei_c)
            for q in range(SB):
                q0 = q * 2 * CH
                for j in range(CH // LANES):
                    dv = ei_c[pl.ds(q0 + CH + j * LANES, LANES)]
                    in_r = jnp.logical_and(dv >= lo, dv < hi)
                    sel_c[pl.ds(j * LANES, LANES)] = jnp.where(
                        in_r, dv - lo, RMAX)
                    if p == 0:
                        gsel_c[pl.ds(j * LANES, LANES)] = dv & 15
                        ssel_c[pl.ds(j * LANES, LANES)] = dv >> 4
                ca = pltpu.async_copy(table.at[ei_c.at[pl.ds(q0, CH)]],
                                      rows_v, sem)
                ca.wait()
                pltpu.sync_copy(rows_v, acc_sp.at[sel_c], add=True)
                if p == 0:
                    pltpu.async_copy(i8_sp.at[gsel_c], rows_v, sem).wait()
                    pltpu.sync_copy(rows_v, deg_sp.at[ssel_c], add=True)
            return carry
        lax.fori_loop(0, NCH // SB, _super, 0)

        plsc.subcore_barrier()
        for k in range(cpt):
            c = sid + k * NS

            @pl.when(c < ndr)
            def _drain_chunk():
                r0 = pl.multiple_of(c * DR, 8)
                g0 = pl.multiple_of(lo + c * DR, 8)
                pltpu.sync_copy(acc_sp.at[pl.ds(r0, DR)], rows_v)
                pltpu.sync_copy(rows_v, acc_out.at[cid, pl.ds(g0, DR)])
        if p == 0:
            @pl.when(sid < DGR // DR)
            def _drain_deg():
                d0 = pl.multiple_of(sid * DR, 8)
                pltpu.sync_copy(deg_sp.at[pl.ds(d0, DR)], rows_v)
                pltpu.sync_copy(rows_v, deg_out.at[cid, pl.ds(d0, DR)])
        plsc.subcore_barrier()

        def _rezero(r, carry):
            for c in range(D // LANES):
                rows_v[r, pl.ds(c * LANES, LANES)] = zeros16
            return carry
        if p + 1 < len(RLO):
            lax.fori_loop(0, CH, _rezero, 0)


def _make_hop():
    mesh = plsc.VectorSubcoreMesh(core_axis_name="c", subcore_axis_name="s")
    return functools.partial(
        pl.kernel,
        out_type=[
            jax.ShapeDtypeStruct((NC, N, D), jnp.float32),
            jax.ShapeDtypeStruct((NC, DGR, D), jnp.float32),
        ],
        mesh=mesh,
        scratch_types=[
            pltpu.VMEM((SB * 2 * CH,), jnp.int32),
            pltpu.VMEM((CH,), jnp.int32),
            pltpu.VMEM((CH,), jnp.int32),
            pltpu.VMEM((CH,), jnp.int32),
            pltpu.VMEM((CH, D), jnp.float32),
            pltpu.VMEM_SHARED((ACCR, D), jnp.float32),
            pltpu.VMEM_SHARED((DGR, D), jnp.float32),
            pltpu.VMEM_SHARED((16, D), jnp.float32),
            pltpu.SemaphoreType.DMA,
        ],
    )(_hop_body)


_BN = 1000


def _combine_body(acc_ref, deg_ref, out_ref):
    s = acc_ref[0] + acc_ref[1]
    dsum = deg_ref[0, :, 0] + deg_ref[1, :, 0]
    out_ref[...] = s / jnp.maximum(dsum, 1.0)[:, None]


def _mlp_body(h1_ref, acc_ref, deg_ref, W1_ref, b1_ref, W2_ref, b2_ref,
              out_ref):
    s = acc_ref[0] + acc_ref[1]
    dsum = deg_ref[0, :, 0] + deg_ref[1, :, 0]
    h2 = s / jnp.maximum(dsum, 1.0)[:, None]
    a = jnp.concatenate([h1_ref[...], h2], axis=1)
    z = jnp.dot(a, W1_ref[...], preferred_element_type=jnp.float32)
    z = jnp.maximum(z + b1_ref[...], 0.0)
    o = jnp.dot(z, W2_ref[...], preferred_element_type=jnp.float32)
    o = o + b2_ref[...]
    nrm = jnp.sqrt(jnp.sum(o * o, axis=1, keepdims=True))
    out_ref[...] = o / jnp.maximum(nrm, 1e-12)


def _unpack_deg(deg_packed):
    d = deg_packed.reshape(NC, DGR, 16, 8)[:, :, :, 0]
    return d.reshape(NC, DGR * 16)[:, :N].reshape(NC, N, 1)


def _interleave(ei):
    sidx = ei[0].reshape(NW, NCH, 1, CH)
    didx = ei[1].reshape(NW, NCH, 1, CH)
    return jnp.concatenate([sidx, didx], axis=2).reshape(-1)


def kernel(x, edge_index1, edge_index2, W1, b1, W2, b2):
    hop = _make_hop()
    i8 = jnp.repeat(jnp.eye(16, dtype=jnp.float32), 8, axis=1)

    acc1, deg1p = hop(x, _interleave(edge_index1), i8)
    deg1 = _unpack_deg(deg1p)

    combine = pl.pallas_call(
        _combine_body,
        grid=(N // _BN,),
        in_specs=[
            pl.BlockSpec((NC, _BN, D), lambda i: (0, i, 0)),
            pl.BlockSpec((NC, _BN, 1), lambda i: (0, i, 0)),
        ],
        out_specs=pl.BlockSpec((_BN, D), lambda i: (i, 0)),
        out_shape=jax.ShapeDtypeStruct((N, D), jnp.float32),
    )
    h1 = combine(acc1, deg1)

    acc2, deg2p = hop(h1, _interleave(edge_index2), i8)
    deg2 = _unpack_deg(deg2p)

    mlp = pl.pallas_call(
        _mlp_body,
        grid=(N // _BN,),
        in_specs=[
            pl.BlockSpec((_BN, D), lambda i: (i, 0)),
            pl.BlockSpec((NC, _BN, D), lambda i: (0, i, 0)),
            pl.BlockSpec((NC, _BN, 1), lambda i: (0, i, 0)),
            pl.BlockSpec((2 * D, HID), lambda i: (0, 0)),
            pl.BlockSpec((1, HID), lambda i: (0, 0)),
            pl.BlockSpec((HID, OUT), lambda i: (0, 0)),
            pl.BlockSpec((1, OUT), lambda i: (0, 0)),
        ],
        out_specs=pl.BlockSpec((_BN, OUT), lambda i: (i, 0)),
        out_shape=jax.ShapeDtypeStruct((N, OUT), jnp.float32),
    )
    return mlp(h1, acc2, deg2, W1, b1.reshape(1, HID), W2, b2.reshape(1, OUT))
